# Initial kernel scaffold; baseline (speedup 1.0000x reference)
import jax
import jax.numpy as jnp
from jax.experimental import pallas as pl


def kernel(adjacency_matrix, beta):
    raise NotImplementedError("write your pallas kernel here")


# speedup vs baseline: 16.2721x; 16.2721x over previous
import numpy as np
import jax
import jax.numpy as jnp
from jax.experimental import pallas as pl
from jax.experimental.pallas import tpu as pltpu

_N = 512
_Q = 10
_QP = 16
_ITERS = 5
_EPS = 1e-12
_NEG = -1e30


def _bp_kernel(W_ref, m0t_ref, psi0_ref, beta_ref,
               psi_out_ref, reg_ref, ent_ref,
               msg_ref, t_ref, ew_ref, S_ref, psi_ref, h_ref):
    beta = beta_ref[0, 0]
    W = W_ref[...]
    mean_w = jnp.sum(W) / (_N * _N)
    bm = beta * mean_w
    ew = jnp.exp(beta * W) - 1.0
    ew_ref[...] = ew

    lane = jax.lax.broadcasted_iota(jnp.int32, (_N, _QP), 1)
    lmask = lane < _Q

    psi0 = psi0_ref[...]
    psi_ref[...] = psi0
    h_ref[...] = -bm * jnp.sum(psi0, axis=0, keepdims=True)

    for k in range(_Q):
        tk = 1.0 + m0t_ref[k] * ew
        t_ref[k] = tk
        S_ref[:, k:k + 1] = jnp.sum(jnp.log(tk), axis=1, keepdims=True)

    def body(it, carry):
        h = h_ref[...]
        G = jnp.where(lmask, S_ref[...] + h, _NEG)
        m = jnp.max(G, axis=1, keepdims=True)
        a = jnp.exp(G - m)
        den = jnp.zeros((_N, _N), jnp.float32)
        for k in range(_Q):
            r = a[:, k:k + 1] * (1.0 / t_ref[k])
            msg_ref[k] = r
            den = den + r
        ewd = ew_ref[...] * (1.0 / jnp.transpose(den))
        for k in range(_Q):
            tk = 1.0 + jnp.transpose(msg_ref[k]) * ewd
            t_ref[k] = tk
            S_ref[:, k:k + 1] = jnp.sum(jnp.log(tk), axis=1, keepdims=True)
        Gh = jnp.where(lmask, S_ref[...] + h, _NEG)
        m2 = jnp.max(Gh, axis=1, keepdims=True)
        e2 = jnp.exp(Gh - m2)
        out = e2 / jnp.sum(e2, axis=1, keepdims=True)
        h_ref[...] = h + bm * (jnp.sum(psi_ref[...], axis=0, keepdims=True)
                               - jnp.sum(out, axis=0, keepdims=True))
        psi_ref[...] = out
        return carry

    jax.lax.fori_loop(0, _ITERS, body, 0)

    psi = psi_ref[...]
    psi_out_ref[...] = psi[:, :_Q]
    colsum = jnp.sum(psi, axis=0, keepdims=True)
    reg = jnp.sum(jnp.square(colsum / _N)) * np.float32(np.sqrt(_Q))
    reg_ref[...] = jnp.reshape(reg, (1, 1))
    ent = -jnp.sum(psi * jnp.log(psi + _EPS)) / (_N * np.float32(np.log(_Q)))
    ent_ref[...] = jnp.reshape(ent, (1, 1))


def _bp_pallas(W, m0t, psi0p, beta11):
    return pl.pallas_call(
        _bp_kernel,
        out_shape=[
            jax.ShapeDtypeStruct((_N, _Q), jnp.float32),
            jax.ShapeDtypeStruct((1, 1), jnp.float32),
            jax.ShapeDtypeStruct((1, 1), jnp.float32),
        ],
        in_specs=[
            pl.BlockSpec(memory_space=pltpu.VMEM),
            pl.BlockSpec(memory_space=pltpu.VMEM),
            pl.BlockSpec(memory_space=pltpu.VMEM),
            pl.BlockSpec(memory_space=pltpu.SMEM),
        ],
        out_specs=[pl.BlockSpec(memory_space=pltpu.VMEM)] * 3,
        scratch_shapes=[
            pltpu.VMEM((_Q, _N, _N), jnp.float32),
            pltpu.VMEM((_Q, _N, _N), jnp.float32),
            pltpu.VMEM((_N, _N), jnp.float32),
            pltpu.VMEM((_N, _QP), jnp.float32),
            pltpu.VMEM((_N, _QP), jnp.float32),
            pltpu.VMEM((1, _QP), jnp.float32),
        ],
    )(W, m0t, psi0p, beta11)


def kernel(adjacency_matrix, beta):
    A = adjacency_matrix.astype(jnp.float32)
    W = 0.5 * (A + A.T)
    W = W * (1.0 - jnp.eye(_N, dtype=W.dtype))
    k1, k2 = jax.random.split(jax.random.key(0))
    psi0 = jax.random.uniform(k1, (_N, _Q), dtype=jnp.float32)
    psi0 = psi0 / psi0.sum(1, keepdims=True)
    E = _N * (_N - 1)
    msg0 = jax.random.uniform(k2, (E, _Q), dtype=jnp.float32)
    msg0 = msg0 / msg0.sum(1, keepdims=True)
    msgr = msg0.reshape(_N, _N - 1, _Q)
    i_idx = jnp.arange(_N)[:, None]
    j_idx = jnp.arange(_N)[None, :]
    c = jnp.where(j_idx > i_idx, j_idx - 1, j_idx)
    c = jnp.minimum(c, _N - 2)
    dense = jnp.take_along_axis(msgr, c[:, :, None], axis=1)
    dense = jnp.where((i_idx == j_idx)[:, :, None], 0.0, dense)
    m0t = dense.transpose(2, 1, 0)
    psi0p = jnp.pad(psi0, ((0, 0), (0, _QP - _Q)))
    beta11 = jnp.reshape(beta.astype(jnp.float32), (1, 1))

    psi, reg, ent = _bp_pallas(W, m0t, psi0p, beta11)

    deg = W.sum(1)
    two_m = W.sum()
    modularity = (jnp.trace(psi.T @ (W @ psi))
                  - jnp.sum(jnp.square(deg @ psi)) / two_m) / two_m
    return (psi, jnp.reshape(reg, ()), jnp.reshape(ent, ()), modularity)


# speedup vs baseline: 25.9516x; 1.5949x over previous
import numpy as np
import jax
import jax.numpy as jnp
from jax.experimental import pallas as pl
from jax.experimental.pallas import tpu as pltpu

_N = 512
_Q = 10
_QP = 16
_ITERS = 5
_EPS = 1e-12
_NEG = -1e30


def _bp_kernel(W_ref, m0t_ref, psi0_ref, beta_ref,
               psi_out_ref, reg_ref, ent_ref,
               msg_ref, t_ref, ew_ref, S_ref, psi_ref, h_ref):
    beta = beta_ref[0, 0]
    W = W_ref[...]
    mean_w = jnp.sum(W) / (_N * _N)
    bm = beta * mean_w
    ew = jnp.exp(beta * W) - 1.0
    ew_ref[...] = ew

    lane = jax.lax.broadcasted_iota(jnp.int32, (_N, _QP), 1)
    lmask = lane < _Q

    psi0 = psi0_ref[...]
    psi_ref[...] = psi0
    h_ref[...] = -bm * jnp.sum(psi0, axis=0, keepdims=True)

    for k in range(_Q):
        tk = 1.0 + jnp.transpose(m0t_ref[k]) * ew
        t_ref[k] = tk
        S_ref[:, k:k + 1] = jnp.sum(jnp.log(tk), axis=1, keepdims=True)

    def body(it, carry):
        h = h_ref[...]
        G = jnp.where(lmask, S_ref[...] + h, _NEG)
        m = jnp.max(G, axis=1, keepdims=True)
        a = jnp.exp(G - m)
        den = jnp.zeros((_N, _N), jnp.float32)
        for k in range(_Q):
            r = a[:, k:k + 1] * (1.0 / t_ref[k])
            msg_ref[k] = r
            den = den + r
        ewd = ew_ref[...] * (1.0 / jnp.transpose(den))
        for k in range(_Q):
            tk = 1.0 + jnp.transpose(msg_ref[k]) * ewd
            t_ref[k] = tk
            S_ref[:, k:k + 1] = jnp.sum(jnp.log(tk), axis=1, keepdims=True)
        Gh = jnp.where(lmask, S_ref[...] + h, _NEG)
        m2 = jnp.max(Gh, axis=1, keepdims=True)
        e2 = jnp.exp(Gh - m2)
        out = e2 / jnp.sum(e2, axis=1, keepdims=True)
        h_ref[...] = h + bm * (jnp.sum(psi_ref[...], axis=0, keepdims=True)
                               - jnp.sum(out, axis=0, keepdims=True))
        psi_ref[...] = out
        return carry

    jax.lax.fori_loop(0, _ITERS, body, 0)

    psi = psi_ref[...]
    psi_out_ref[...] = psi[:, :_Q]
    colsum = jnp.sum(psi, axis=0, keepdims=True)
    reg = jnp.sum(jnp.square(colsum / _N)) * np.float32(np.sqrt(_Q))
    reg_ref[...] = jnp.reshape(reg, (1, 1))
    ent = -jnp.sum(psi * jnp.log(psi + _EPS)) / (_N * np.float32(np.log(_Q)))
    ent_ref[...] = jnp.reshape(ent, (1, 1))


def _bp_pallas(W, m0t, psi0p, beta11):
    return pl.pallas_call(
        _bp_kernel,
        out_shape=[
            jax.ShapeDtypeStruct((_N, _Q), jnp.float32),
            jax.ShapeDtypeStruct((1, 1), jnp.float32),
            jax.ShapeDtypeStruct((1, 1), jnp.float32),
        ],
        in_specs=[
            pl.BlockSpec(memory_space=pltpu.VMEM),
            pl.BlockSpec(memory_space=pltpu.VMEM),
            pl.BlockSpec(memory_space=pltpu.VMEM),
            pl.BlockSpec(memory_space=pltpu.SMEM),
        ],
        out_specs=[pl.BlockSpec(memory_space=pltpu.VMEM)] * 3,
        scratch_shapes=[
            pltpu.VMEM((_Q, _N, _N), jnp.float32),
            pltpu.VMEM((_Q, _N, _N), jnp.float32),
            pltpu.VMEM((_N, _N), jnp.float32),
            pltpu.VMEM((_N, _QP), jnp.float32),
            pltpu.VMEM((_N, _QP), jnp.float32),
            pltpu.VMEM((1, _QP), jnp.float32),
        ],
    )(W, m0t, psi0p, beta11)


def kernel(adjacency_matrix, beta):
    A = adjacency_matrix.astype(jnp.float32)
    W = 0.5 * (A + A.T)
    W = W * (1.0 - jnp.eye(_N, dtype=W.dtype))
    k1, k2 = jax.random.split(jax.random.key(0))
    psi0 = jax.random.uniform(k1, (_N, _Q), dtype=jnp.float32)
    psi0 = psi0 / psi0.sum(1, keepdims=True)
    E = _N * (_N - 1)
    msg0 = jax.random.uniform(k2, (E, _Q), dtype=jnp.float32)
    msg0 = msg0 / msg0.sum(1, keepdims=True)
    P = msg0.T
    C = jnp.concatenate(
        [jnp.zeros((_Q, _N - 1, 1), jnp.float32),
         P.reshape(_Q, _N - 1, _N)], axis=2)
    Yf = jnp.concatenate(
        [C.reshape(_Q, (_N - 1) * (_N + 1)),
         jnp.zeros((_Q, 1), jnp.float32)], axis=1)
    m0t = Yf.reshape(_Q, _N, _N)
    psi0p = jnp.pad(psi0, ((0, 0), (0, _QP - _Q)))
    beta11 = jnp.reshape(beta.astype(jnp.float32), (1, 1))

    psi, reg, ent = _bp_pallas(W, m0t, psi0p, beta11)

    deg = W.sum(1)
    two_m = W.sum()
    modularity = (jnp.trace(psi.T @ (W @ psi))
                  - jnp.sum(jnp.square(deg @ psi)) / two_m) / two_m
    return (psi, jnp.reshape(reg, ()), jnp.reshape(ent, ()), modularity)


# speedup vs baseline: 46.0486x; 1.7744x over previous
import numpy as np
import jax
import jax.extend
import jax.numpy as jnp
from jax.experimental import pallas as pl
from jax.experimental.pallas import tpu as pltpu

_N = 512
_Q = 10
_QP = 16
_ITERS = 5
_EPS = 1e-12
_NEG = -1e30


def _bp_kernel(W_ref, m0t_ref, psi0_ref, beta_ref,
               psi_out_ref, reg_ref, ent_ref,
               msg_ref, t_ref, ew_ref, S_ref, psi_ref, h_ref):
    beta = beta_ref[0, 0]
    W = W_ref[...]
    mean_w = jnp.sum(W) / (_N * _N)
    bm = beta * mean_w
    ew = jnp.exp(beta * W) - 1.0
    ew_ref[...] = ew

    lane = jax.lax.broadcasted_iota(jnp.int32, (_N, _QP), 1)
    lmask = lane < _Q

    psi0 = psi0_ref[...]
    psi_ref[...] = psi0
    h_ref[...] = -bm * jnp.sum(psi0, axis=0, keepdims=True)

    for k in range(_Q):
        tk = 1.0 + m0t_ref[k] * ew
        t_ref[k] = tk
        S_ref[:, k:k + 1] = jnp.sum(jnp.log(tk), axis=1, keepdims=True)

    def body(it, carry):
        h = h_ref[...]
        G = jnp.where(lmask, S_ref[...] + h, _NEG)
        m = jnp.max(G, axis=1, keepdims=True)
        a = jnp.exp(G - m)
        den = jnp.zeros((_N, _N), jnp.float32)
        for k in range(_Q):
            r = a[:, k:k + 1] * (1.0 / t_ref[k])
            msg_ref[k] = r
            den = den + r
        ewd = ew_ref[...] * (1.0 / jnp.transpose(den))
        for k in range(_Q):
            tk = 1.0 + jnp.transpose(msg_ref[k]) * ewd
            t_ref[k] = tk
            S_ref[:, k:k + 1] = jnp.sum(jnp.log(tk), axis=1, keepdims=True)
        Gh = jnp.where(lmask, S_ref[...] + h, _NEG)
        m2 = jnp.max(Gh, axis=1, keepdims=True)
        e2 = jnp.exp(Gh - m2)
        out = e2 / jnp.sum(e2, axis=1, keepdims=True)
        h_ref[...] = h + bm * (jnp.sum(psi_ref[...], axis=0, keepdims=True)
                               - jnp.sum(out, axis=0, keepdims=True))
        psi_ref[...] = out
        return carry

    jax.lax.fori_loop(0, _ITERS, body, 0)

    psi = psi_ref[...]
    psi_out_ref[...] = psi[:, :_Q]
    colsum = jnp.sum(psi, axis=0, keepdims=True)
    reg = jnp.sum(jnp.square(colsum / _N)) * np.float32(np.sqrt(_Q))
    reg_ref[...] = jnp.reshape(reg, (1, 1))
    ent = -jnp.sum(psi * jnp.log(psi + _EPS)) / (_N * np.float32(np.log(_Q)))
    ent_ref[...] = jnp.reshape(ent, (1, 1))


def _bp_pallas(W, m0t, psi0p, beta11):
    return pl.pallas_call(
        _bp_kernel,
        out_shape=[
            jax.ShapeDtypeStruct((_N, _Q), jnp.float32),
            jax.ShapeDtypeStruct((1, 1), jnp.float32),
            jax.ShapeDtypeStruct((1, 1), jnp.float32),
        ],
        in_specs=[
            pl.BlockSpec(memory_space=pltpu.VMEM),
            pl.BlockSpec(memory_space=pltpu.VMEM),
            pl.BlockSpec(memory_space=pltpu.VMEM),
            pl.BlockSpec(memory_space=pltpu.SMEM),
        ],
        out_specs=[pl.BlockSpec(memory_space=pltpu.VMEM)] * 3,
        scratch_shapes=[
            pltpu.VMEM((_Q, _N, _N), jnp.float32),
            pltpu.VMEM((_Q, _N, _N), jnp.float32),
            pltpu.VMEM((_N, _N), jnp.float32),
            pltpu.VMEM((_N, _QP), jnp.float32),
            pltpu.VMEM((_N, _QP), jnp.float32),
            pltpu.VMEM((1, _QP), jnp.float32),
        ],
    )(W, m0t, psi0p, beta11)


def kernel(adjacency_matrix, beta):
    A = adjacency_matrix.astype(jnp.float32)
    W = 0.5 * (A + A.T)
    W = W * (1.0 - jnp.eye(_N, dtype=W.dtype))
    k1, k2 = jax.random.split(jax.random.key(0))
    psi0 = jax.random.uniform(k1, (_N, _Q), dtype=jnp.float32)
    psi0 = psi0 / psi0.sum(1, keepdims=True)
    kd2 = jax.random.key_data(k2)
    ii = jnp.arange(_N, dtype=jnp.uint32)[:, None]
    jj = jnp.arange(_N, dtype=jnp.uint32)[None, :]
    e_id = jj * jnp.uint32(_N - 1) + ii - (ii > jj).astype(jnp.uint32)
    p_cnt = (e_id * jnp.uint32(_Q))[None, :, :] \
        + jnp.arange(_Q, dtype=jnp.uint32)[:, None, None]
    b1, b2 = jax.extend.random.threefry2x32_p.bind(
        kd2[0], kd2[1], jnp.zeros_like(p_cnt), p_cnt)
    bits = b1 ^ b2
    fbits = (bits >> jnp.uint32(9)) | jnp.uint32(0x3F800000)
    raw = jax.lax.bitcast_convert_type(fbits, jnp.float32) - 1.0
    m0t = raw / jnp.sum(raw, axis=0, keepdims=True)
    psi0p = jnp.pad(psi0, ((0, 0), (0, _QP - _Q)))
    beta11 = jnp.reshape(beta.astype(jnp.float32), (1, 1))

    psi, reg, ent = _bp_pallas(W, m0t, psi0p, beta11)

    deg = W.sum(1)
    two_m = W.sum()
    modularity = (jnp.trace(psi.T @ (W @ psi))
                  - jnp.sum(jnp.square(deg @ psi)) / two_m) / two_m
    return (psi, jnp.reshape(reg, ()), jnp.reshape(ent, ()), modularity)
